# Initial kernel scaffold; baseline (speedup 1.0000x reference)
import jax
import jax.numpy as jnp
from jax.experimental import pallas as pl


def kernel(node_features, edge_features, global_features, edge_index, W_e1, b_e1, W_n1, W_in1, b_n1, W_e2, W_ge2, b_e2, W_n2, W_in2, W_gn2, b_n2, W_gn, W_gedge, W_gg, b_g, W_mean, b_mean, W_logstd, b_logstd):
    raise NotImplementedError("write your pallas kernel here")


# speedup vs baseline: 3.1465x; 3.1465x over previous
import functools

import jax
import jax.numpy as jnp
from jax import lax
from jax.experimental import pallas as pl
from jax.experimental.pallas import tpu as pltpu
from jax.experimental.pallas import tpu_sc as plsc

_N_NODES = 10000
_N_EDGES = 320000
_B_E = 4000
_B_N = 2000
_W = 200
_SUB = 100
_NSUB = _W // _SUB
_N_WIN = _N_EDGES // _W
_TILES = 16
_N_PAD = 10240
_ROWS_PT = _N_PAD // _TILES


def _edge_body(ef_ref, g_ref, we1_ref, be1_ref, we2_ref, wge2_ref, be2_ref,
               e1_ref, e2_ref, esum_ref):
    e1 = jnp.maximum(ef_ref[...] @ we1_ref[...] + be1_ref[...], 0.0)
    gterm = g_ref[...] @ wge2_ref[...] + be2_ref[...]
    e2 = jnp.maximum(e1 @ we2_ref[...] + gterm, 0.0)
    e1_ref[...] = e1
    e2_ref[...] = e2

    @pl.when(pl.program_id(0) == 0)
    def _():
        esum_ref[...] = jnp.zeros_like(esum_ref)

    esum_ref[...] += jnp.sum(e2, axis=0, keepdims=True)


def _edge_pass(ef, g, we1, be1, we2, wge2, be2, *, interpret=False):
    n_blk = _N_EDGES // _B_E
    return pl.pallas_call(
        _edge_body,
        grid=(n_blk,),
        in_specs=[
            pl.BlockSpec((_B_E, 16), lambda i: (i, 0)),
            pl.BlockSpec((1, 32), lambda i: (0, 0)),
            pl.BlockSpec((16, 256), lambda i: (0, 0)),
            pl.BlockSpec((1, 256), lambda i: (0, 0)),
            pl.BlockSpec((256, 128), lambda i: (0, 0)),
            pl.BlockSpec((32, 128), lambda i: (0, 0)),
            pl.BlockSpec((1, 128), lambda i: (0, 0)),
        ],
        out_specs=[
            pl.BlockSpec((_B_E, 256), lambda i: (i, 0)),
            pl.BlockSpec((_B_E, 128), lambda i: (i, 0)),
            pl.BlockSpec((1, 128), lambda i: (0, 0)),
        ],
        out_shape=[
            jax.ShapeDtypeStruct((_N_EDGES, 256), jnp.float32),
            jax.ShapeDtypeStruct((_N_EDGES, 128), jnp.float32),
            jax.ShapeDtypeStruct((1, 128), jnp.float32),
        ],
        interpret=interpret,
    )(ef, g, we1, be1, we2, wge2, be2)


def _sc_agg_body(e1_hbm, e2_hbm, recv_hbm, zeros_hbm, zeros1_hbm, ones_hbm,
                 agg1_out, cnt_out, agg2_out,
                 acc, cntacc, upd, idx, ones_v):
    cid = lax.axis_index("c")
    sid = lax.axis_index("s")
    r0 = sid * _ROWS_PT
    col0 = cid * 128

    pltpu.sync_copy(zeros_hbm.at[pl.ds(r0, _ROWS_PT), :],
                    acc.at[pl.ds(r0, _ROWS_PT), :])

    @pl.when(cid == 0)
    def _():
        pltpu.sync_copy(zeros1_hbm.at[pl.ds(r0, _ROWS_PT)],
                        cntacc.at[pl.ds(r0, _ROWS_PT)])

    pltpu.sync_copy(ones_hbm, ones_v)
    plsc.subcore_barrier()

    win_pt_a = _N_WIN // _TILES

    def _win_a(w, carry):
        wr = sid * win_pt_a + w
        e0 = wr * _W
        pltpu.sync_copy(recv_hbm.at[wr], idx)
        pltpu.sync_copy(e1_hbm.at[pl.ds(e0, _W), pl.ds(col0, 128)], upd)
        for j in range(_NSUB):
            pltpu.sync_copy(upd.at[pl.ds(j * _SUB, _SUB), :],
                            acc.at[idx.at[j]], add=True)

        @pl.when(cid == 0)
        def _():
            for j in range(_NSUB):
                pltpu.sync_copy(ones_v, cntacc.at[idx.at[j]], add=True)

        return carry

    lax.fori_loop(0, win_pt_a, _win_a, 0)
    plsc.subcore_barrier()

    pltpu.sync_copy(acc.at[pl.ds(r0, _ROWS_PT), :],
                    agg1_out.at[pl.ds(r0, _ROWS_PT), pl.ds(col0, 128)])

    @pl.when(cid == 0)
    def _():
        pltpu.sync_copy(cntacc.at[pl.ds(r0, _ROWS_PT)],
                        cnt_out.at[pl.ds(r0, _ROWS_PT)])

    pltpu.sync_copy(zeros_hbm.at[pl.ds(r0, _ROWS_PT), :],
                    acc.at[pl.ds(r0, _ROWS_PT), :])
    plsc.subcore_barrier()

    win_pt_b = _N_WIN // (2 * _TILES)

    def _win_b(w, carry):
        gw = cid * (_N_WIN // 2) + sid * win_pt_b + w
        e0 = gw * _W
        pltpu.sync_copy(recv_hbm.at[gw], idx)
        pltpu.sync_copy(e2_hbm.at[pl.ds(e0, _W), :], upd)
        for j in range(_NSUB):
            pltpu.sync_copy(upd.at[pl.ds(j * _SUB, _SUB), :],
                            acc.at[idx.at[j]], add=True)
        return carry

    lax.fori_loop(0, win_pt_b, _win_b, 0)
    plsc.subcore_barrier()
    pltpu.sync_copy(acc.at[pl.ds(r0, _ROWS_PT), :],
                    agg2_out.at[cid, pl.ds(r0, _ROWS_PT), :])


def _sc_aggregate(e1, e2, recv3, zeros_n, zeros1, ones_h):
    agg = pl.kernel(
        _sc_agg_body,
        out_type=[
            jax.ShapeDtypeStruct((_N_PAD, 256), jnp.float32),
            jax.ShapeDtypeStruct((_N_PAD,), jnp.float32),
            jax.ShapeDtypeStruct((2, _N_PAD, 128), jnp.float32),
        ],
        mesh=plsc.VectorSubcoreMesh(core_axis_name="c", subcore_axis_name="s"),
        scratch_types=[
            pltpu.VMEM_SHARED((_N_PAD, 128), jnp.float32),
            pltpu.VMEM_SHARED((_N_PAD,), jnp.float32),
            pltpu.VMEM((_W, 128), jnp.float32),
            pltpu.VMEM((_NSUB, _SUB), jnp.int32),
            pltpu.VMEM((_SUB,), jnp.float32),
        ],
    )
    return agg(e1, e2, recv3, zeros_n, zeros1, ones_h)


def _node_body(nf_ref, a1_ref, cnt_ref, p0_ref, p1_ref, g_ref,
               wn1_ref, win1_ref, bn1_ref, wn2_ref, win2_ref, wgn2_ref,
               bn2_ref, nsum_ref):
    cnt = jnp.maximum(cnt_ref[...], 1.0)
    agg1 = a1_ref[...] / cnt
    agg2 = (p0_ref[...] + p1_ref[...]) / cnt
    n1 = jnp.maximum(nf_ref[...] @ wn1_ref[...] + agg1 @ win1_ref[...]
                     + bn1_ref[...], 0.0)
    gterm = g_ref[...] @ wgn2_ref[...] + bn2_ref[...]
    n2 = jnp.maximum(n1 @ wn2_ref[...] + agg2 @ win2_ref[...] + gterm, 0.0)

    @pl.when(pl.program_id(0) == 0)
    def _():
        nsum_ref[...] = jnp.zeros_like(nsum_ref)

    nsum_ref[...] += jnp.sum(n2, axis=0, keepdims=True)


def _node_pass(nf, a1, cnt, p0, p1, g, wn1, win1, bn1, wn2, win2, wgn2, bn2,
               *, interpret=False):
    n_blk = _N_NODES // _B_N
    return pl.pallas_call(
        _node_body,
        grid=(n_blk,),
        in_specs=[
            pl.BlockSpec((_B_N, 128), lambda i: (i, 0)),
            pl.BlockSpec((_B_N, 256), lambda i: (i, 0)),
            pl.BlockSpec((_B_N, 1), lambda i: (i, 0)),
            pl.BlockSpec((_B_N, 128), lambda i: (i, 0)),
            pl.BlockSpec((_B_N, 128), lambda i: (i, 0)),
            pl.BlockSpec((1, 32), lambda i: (0, 0)),
            pl.BlockSpec((128, 256), lambda i: (0, 0)),
            pl.BlockSpec((256, 256), lambda i: (0, 0)),
            pl.BlockSpec((1, 256), lambda i: (0, 0)),
            pl.BlockSpec((256, 128), lambda i: (0, 0)),
            pl.BlockSpec((128, 128), lambda i: (0, 0)),
            pl.BlockSpec((32, 128), lambda i: (0, 0)),
            pl.BlockSpec((1, 128), lambda i: (0, 0)),
        ],
        out_specs=pl.BlockSpec((1, 128), lambda i: (0, 0)),
        out_shape=jax.ShapeDtypeStruct((1, 128), jnp.float32),
        interpret=interpret,
    )(nf, a1, cnt, p0, p1, g, wn1, win1, bn1, wn2, win2, wgn2, bn2)


def _head_body(nsum_ref, esum_ref, g_ref, wgn_ref, wge_ref, wgg_ref, bg_ref,
               wm_ref, bm_ref, wl_ref, bl_ref, mean_ref, logstd_ref):
    u = (nsum_ref[...] * (1.0 / _N_NODES)) @ wgn_ref[...] \
        + (esum_ref[...] * (1.0 / _N_EDGES)) @ wge_ref[...] \
        + g_ref[...] @ wgg_ref[...] + bg_ref[...]
    gv = jnp.maximum(u, 0.0)
    mean_ref[...] = gv @ wm_ref[...] + bm_ref[...]
    logstd_ref[...] = jnp.clip(gv @ wl_ref[...] + bl_ref[...], -20.0, 2.0)


def _head_pass(nsum, esum, g, wgn, wge, wgg, bg, wm, bm, wl, bl,
               *, interpret=False):
    return pl.pallas_call(
        _head_body,
        out_shape=[
            jax.ShapeDtypeStruct((1, 8), jnp.float32),
            jax.ShapeDtypeStruct((1, 8), jnp.float32),
        ],
        interpret=interpret,
    )(nsum, esum, g, wgn, wge, wgg, bg, wm, bm, wl, bl)


def kernel(node_features, edge_features, global_features, edge_index,
           W_e1, b_e1, W_n1, W_in1, b_n1,
           W_e2, W_ge2, b_e2,
           W_n2, W_in2, W_gn2, b_n2,
           W_gn, W_gedge, W_gg, b_g,
           W_mean, b_mean, W_logstd, b_logstd):
    recv3 = edge_index[1].astype(jnp.int32).reshape(_N_WIN, _NSUB, _SUB)
    zeros_n = jnp.zeros((_N_PAD, 128), jnp.float32)
    zeros1 = jnp.zeros((_N_PAD,), jnp.float32)
    ones_h = jnp.ones((_SUB,), jnp.float32)

    e1, e2, esum = _edge_pass(
        edge_features, global_features, W_e1, b_e1.reshape(1, -1),
        W_e2, W_ge2, b_e2.reshape(1, -1))
    agg1s, cnt1, agg2p = _sc_aggregate(e1, e2, recv3, zeros_n, zeros1, ones_h)
    cnt2 = cnt1.reshape(_N_PAD, 1)
    nsum = _node_pass(
        node_features, agg1s, cnt2, agg2p[0], agg2p[1], global_features,
        W_n1, W_in1, b_n1.reshape(1, -1), W_n2, W_in2, W_gn2,
        b_n2.reshape(1, -1))
    return _head_pass(
        nsum, esum, global_features, W_gn, W_gedge, W_gg, b_g.reshape(1, -1),
        W_mean, b_mean.reshape(1, -1), W_logstd, b_logstd.reshape(1, -1))


# speedup vs baseline: 4.0112x; 1.2748x over previous
import jax
import jax.numpy as jnp
from jax import lax
from jax.experimental import pallas as pl
from jax.experimental.pallas import tpu as pltpu
from jax.experimental.pallas import tpu_sc as plsc

_N_NODES = 10000
_N_EDGES = 320000
_C = 128
_N_EPAD = 327680
_N_CH = _N_EPAD // _C
_B_E = 4096
_B_N = 2000
_TILES = 16
_N_PAD = 10240
_ROWS_PT = _N_PAD // _TILES
_CH_A_PT = _N_CH // _TILES
_IDXB = 80
_CH_B_PT = _N_CH // (2 * _TILES)


def _edge_body(ef_ref, g_ref, we1_ref, be1_ref, we2_ref, wge2_ref, be2_ref,
               e1_ref, e2_ref, esum_ref):
    e1 = jnp.maximum(ef_ref[...] @ we1_ref[...] + be1_ref[...], 0.0)
    gterm = g_ref[...] @ wge2_ref[...] + be2_ref[...]
    e2 = jnp.maximum(e1 @ we2_ref[...] + gterm, 0.0)
    e1_ref[...] = e1
    e2_ref[...] = e2

    @pl.when(pl.program_id(0) == 0)
    def _():
        esum_ref[...] = jnp.zeros_like(esum_ref)

    rid = pl.program_id(0) * _B_E + lax.broadcasted_iota(
        jnp.int32, (_B_E, 1), 0)
    mvec = jnp.where(rid < _N_EDGES, 1.0, 0.0)
    esum_ref[...] += jnp.sum(e2 * mvec, axis=0, keepdims=True)


def _edge_pass(ef, g, we1, be1, we2, wge2, be2, *, interpret=False):
    n_blk = _N_EPAD // _B_E
    return pl.pallas_call(
        _edge_body,
        grid=(n_blk,),
        in_specs=[
            pl.BlockSpec((_B_E, 16), lambda i: (i, 0)),
            pl.BlockSpec((1, 32), lambda i: (0, 0)),
            pl.BlockSpec((16, 256), lambda i: (0, 0)),
            pl.BlockSpec((1, 256), lambda i: (0, 0)),
            pl.BlockSpec((256, 128), lambda i: (0, 0)),
            pl.BlockSpec((32, 128), lambda i: (0, 0)),
            pl.BlockSpec((1, 128), lambda i: (0, 0)),
        ],
        out_specs=[
            pl.BlockSpec((_B_E, 256), lambda i: (i, 0)),
            pl.BlockSpec((_B_E, 128), lambda i: (i, 0)),
            pl.BlockSpec((1, 128), lambda i: (0, 0)),
        ],
        out_shape=[
            jax.ShapeDtypeStruct((_N_EPAD, 256), jnp.float32),
            jax.ShapeDtypeStruct((_N_EPAD, 128), jnp.float32),
            jax.ShapeDtypeStruct((1, 128), jnp.float32),
        ],
        interpret=interpret,
    )(ef, g, we1, be1, we2, wge2, be2)


def _sc_agg_body(e1_hbm, e2_hbm, recv_hbm, zeros_hbm, zeros1_hbm, ones_hbm,
                 agg1_out, cnt_out, agg2_out,
                 acc, cntacc, upd, idxb, ones_v, sem0, sem1):
    cid = lax.axis_index("c")
    sid = lax.axis_index("s")
    r0 = sid * _ROWS_PT
    col0 = cid * 128

    pltpu.sync_copy(zeros_hbm.at[pl.ds(r0, _ROWS_PT), :],
                    acc.at[pl.ds(r0, _ROWS_PT), :])

    @pl.when(cid == 0)
    def _():
        pltpu.sync_copy(zeros1_hbm.at[pl.ds(r0, _ROWS_PT)],
                        cntacc.at[pl.ds(r0, _ROWS_PT)])

    pltpu.sync_copy(ones_hbm, ones_v)
    plsc.subcore_barrier()

    def _run_phase(src_slice, n_idx_blocks, row_base, with_counts):
        def _start(ch, b):
            pltpu.async_copy(src_slice(ch), upd.at[b], sem0 if b == 0 else sem1)

        def _wait(b):
            pltpu.make_async_copy(src_slice(0), upd.at[b],
                                  sem0 if b == 0 else sem1).wait()

        def _scatter(c, b):
            pltpu.sync_copy(upd.at[b], acc.at[idxb.at[c]], add=True)
            if with_counts:
                @pl.when(cid == 0)
                def _():
                    pltpu.sync_copy(ones_v, cntacc.at[idxb.at[c]], add=True)

        for blk in range(n_idx_blocks):
            row0 = row_base + blk * _IDXB
            pltpu.sync_copy(recv_hbm.at[pl.ds(row0, _IDXB), :], idxb)
            _start(row0, 0)

            def _pair(i, carry):
                c0 = 2 * i
                c1 = c0 + 1
                _start(row0 + c1, 1)
                _wait(0)
                _scatter(c0, 0)

                @pl.when(c0 + 2 < _IDXB)
                def _():
                    _start(row0 + c0 + 2, 0)

                _wait(1)
                _scatter(c1, 1)
                return carry

            lax.fori_loop(0, _IDXB // 2, _pair, 0)

    _run_phase(
        lambda ch: e1_hbm.at[pl.ds(ch * _C, _C), pl.ds(col0, 128)],
        _CH_A_PT // _IDXB, sid * _CH_A_PT, True)
    plsc.subcore_barrier()

    pltpu.sync_copy(acc.at[pl.ds(r0, _ROWS_PT), :],
                    agg1_out.at[pl.ds(r0, _ROWS_PT), pl.ds(col0, 128)])

    @pl.when(cid == 0)
    def _():
        pltpu.sync_copy(cntacc.at[pl.ds(r0, _ROWS_PT)],
                        cnt_out.at[pl.ds(r0, _ROWS_PT)])

    pltpu.sync_copy(zeros_hbm.at[pl.ds(r0, _ROWS_PT), :],
                    acc.at[pl.ds(r0, _ROWS_PT), :])
    plsc.subcore_barrier()

    _run_phase(
        lambda ch: e2_hbm.at[pl.ds(ch * _C, _C), :],
        _CH_B_PT // _IDXB, cid * (_N_CH // 2) + sid * _CH_B_PT, False)
    plsc.subcore_barrier()
    pltpu.sync_copy(acc.at[pl.ds(r0, _ROWS_PT), :],
                    agg2_out.at[cid, pl.ds(r0, _ROWS_PT), :])


def _sc_aggregate(e1, e2, recv2, zeros_n, zeros1, ones_h):
    agg = pl.kernel(
        _sc_agg_body,
        out_type=[
            jax.ShapeDtypeStruct((_N_PAD, 256), jnp.float32),
            jax.ShapeDtypeStruct((_N_PAD,), jnp.float32),
            jax.ShapeDtypeStruct((2, _N_PAD, 128), jnp.float32),
        ],
        mesh=plsc.VectorSubcoreMesh(core_axis_name="c", subcore_axis_name="s"),
        scratch_types=[
            pltpu.VMEM_SHARED((_N_PAD, 128), jnp.float32),
            pltpu.VMEM_SHARED((_N_PAD,), jnp.float32),
            pltpu.VMEM((2, _C, 128), jnp.float32),
            pltpu.VMEM((_IDXB, _C), jnp.int32),
            pltpu.VMEM((_C,), jnp.float32),
            pltpu.SemaphoreType.DMA,
            pltpu.SemaphoreType.DMA,
        ],
    )
    return agg(e1, e2, recv2, zeros_n, zeros1, ones_h)


def _node_body(nf_ref, a1_ref, cnt_ref, p0_ref, p1_ref, g_ref,
               wn1_ref, win1_ref, bn1_ref, wn2_ref, win2_ref, wgn2_ref,
               bn2_ref, nsum_ref):
    cnt = jnp.maximum(cnt_ref[...], 1.0)
    agg1 = a1_ref[...] / cnt
    agg2 = (p0_ref[...] + p1_ref[...]) / cnt
    n1 = jnp.maximum(nf_ref[...] @ wn1_ref[...] + agg1 @ win1_ref[...]
                     + bn1_ref[...], 0.0)
    gterm = g_ref[...] @ wgn2_ref[...] + bn2_ref[...]
    n2 = jnp.maximum(n1 @ wn2_ref[...] + agg2 @ win2_ref[...] + gterm, 0.0)

    @pl.when(pl.program_id(0) == 0)
    def _():
        nsum_ref[...] = jnp.zeros_like(nsum_ref)

    nsum_ref[...] += jnp.sum(n2, axis=0, keepdims=True)


def _node_pass(nf, a1, cnt, p0, p1, g, wn1, win1, bn1, wn2, win2, wgn2, bn2,
               *, interpret=False):
    n_blk = _N_NODES // _B_N
    return pl.pallas_call(
        _node_body,
        grid=(n_blk,),
        in_specs=[
            pl.BlockSpec((_B_N, 128), lambda i: (i, 0)),
            pl.BlockSpec((_B_N, 256), lambda i: (i, 0)),
            pl.BlockSpec((_B_N, 1), lambda i: (i, 0)),
            pl.BlockSpec((_B_N, 128), lambda i: (i, 0)),
            pl.BlockSpec((_B_N, 128), lambda i: (i, 0)),
            pl.BlockSpec((1, 32), lambda i: (0, 0)),
            pl.BlockSpec((128, 256), lambda i: (0, 0)),
            pl.BlockSpec((256, 256), lambda i: (0, 0)),
            pl.BlockSpec((1, 256), lambda i: (0, 0)),
            pl.BlockSpec((256, 128), lambda i: (0, 0)),
            pl.BlockSpec((128, 128), lambda i: (0, 0)),
            pl.BlockSpec((32, 128), lambda i: (0, 0)),
            pl.BlockSpec((1, 128), lambda i: (0, 0)),
        ],
        out_specs=pl.BlockSpec((1, 128), lambda i: (0, 0)),
        out_shape=jax.ShapeDtypeStruct((1, 128), jnp.float32),
        interpret=interpret,
    )(nf, a1, cnt, p0, p1, g, wn1, win1, bn1, wn2, win2, wgn2, bn2)


def _head_body(nsum_ref, esum_ref, g_ref, wgn_ref, wge_ref, wgg_ref, bg_ref,
               wm_ref, bm_ref, wl_ref, bl_ref, mean_ref, logstd_ref):
    u = (nsum_ref[...] * (1.0 / _N_NODES)) @ wgn_ref[...] \
        + (esum_ref[...] * (1.0 / _N_EDGES)) @ wge_ref[...] \
        + g_ref[...] @ wgg_ref[...] + bg_ref[...]
    gv = jnp.maximum(u, 0.0)
    mean_ref[...] = gv @ wm_ref[...] + bm_ref[...]
    logstd_ref[...] = jnp.clip(gv @ wl_ref[...] + bl_ref[...], -20.0, 2.0)


def _head_pass(nsum, esum, g, wgn, wge, wgg, bg, wm, bm, wl, bl,
               *, interpret=False):
    return pl.pallas_call(
        _head_body,
        out_shape=[
            jax.ShapeDtypeStruct((1, 8), jnp.float32),
            jax.ShapeDtypeStruct((1, 8), jnp.float32),
        ],
        interpret=interpret,
    )(nsum, esum, g, wgn, wge, wgg, bg, wm, bm, wl, bl)


def kernel(node_features, edge_features, global_features, edge_index,
           W_e1, b_e1, W_n1, W_in1, b_n1,
           W_e2, W_ge2, b_e2,
           W_n2, W_in2, W_gn2, b_n2,
           W_gn, W_gedge, W_gg, b_g,
           W_mean, b_mean, W_logstd, b_logstd):
    n_pad_e = _N_EPAD - _N_EDGES
    recv = edge_index[1].astype(jnp.int32)
    pad_idx = _N_NODES + (jnp.arange(n_pad_e, dtype=jnp.int32)
                          % (_N_PAD - _N_NODES))
    recv2 = jnp.concatenate([recv, pad_idx]).reshape(_N_CH, _C)
    ef_pad = jnp.concatenate(
        [edge_features, jnp.zeros((n_pad_e, 16), jnp.float32)], axis=0)
    zeros_n = jnp.zeros((_N_PAD, 128), jnp.float32)
    zeros1 = jnp.zeros((_N_PAD,), jnp.float32)
    ones_h = jnp.ones((_C,), jnp.float32)

    e1, e2, esum = _edge_pass(
        ef_pad, global_features, W_e1, b_e1.reshape(1, -1),
        W_e2, W_ge2, b_e2.reshape(1, -1))
    agg1s, cnt1, agg2p = _sc_aggregate(e1, e2, recv2, zeros_n, zeros1, ones_h)
    cnt2 = cnt1.reshape(_N_PAD, 1)
    nsum = _node_pass(
        node_features, agg1s, cnt2, agg2p[0], agg2p[1], global_features,
        W_n1, W_in1, b_n1.reshape(1, -1), W_n2, W_in2, W_gn2,
        b_n2.reshape(1, -1))
    return _head_pass(
        nsum, esum, global_features, W_gn, W_gedge, W_gg, b_g.reshape(1, -1),
        W_mean, b_mean, W_logstd, b_logstd.reshape(1, -1))
